# baseline (speedup 1.0000x reference)
import functools

import jax
import jax.numpy as jnp
from jax import lax
from jax.experimental import pallas as pl
from jax.experimental.pallas import tpu as pltpu
from jax.experimental.pallas import tpu_sc as plsc

_BLK = 512
_NC = 2
_NS = 16
_LANES = 16


def _vq_block(emb_ref, cb_ref, idx_ref, alpha_ref):
    d = emb_ref.shape[1]
    e = emb_ref[...]
    c = cb_ref[...]
    k = c.shape[0]

    norms = jnp.sum(c * c, axis=1)
    dots = jax.lax.dot_general(
        e, c, (((1,), (1,)), ((), ())),
        preferred_element_type=jnp.float32,
        precision=jax.lax.Precision.HIGHEST)
    alpha = dots / norms[None, :]
    score = dots * alpha

    m = jnp.max(score, axis=1, keepdims=True)
    kiota = jax.lax.broadcasted_iota(jnp.int32, score.shape, 1)
    idx = jnp.min(jnp.where(score == m, kiota, k), axis=1)

    sel = kiota == idx[:, None]
    alpha_sel = jnp.max(jnp.where(sel, alpha, -jnp.inf), axis=1)

    idx_ref[0, 0, :] = idx
    alpha_ref[...] = jnp.broadcast_to(alpha_sel[:, None], (alpha_sel.shape[0], d))


def _tc_search(embedding, codebook):
    b, d = embedding.shape
    k = codebook.shape[0]
    nblk = b // _BLK
    idx, alpha = pl.pallas_call(
        _vq_block,
        grid=(nblk,),
        in_specs=[
            pl.BlockSpec((_BLK, d), lambda i: (i, 0)),
            pl.BlockSpec((k, d), lambda i: (0, 0)),
        ],
        out_specs=[
            pl.BlockSpec((1, 1, _BLK), lambda i: (i, 0, 0)),
            pl.BlockSpec((_BLK, d), lambda i: (i, 0)),
        ],
        out_shape=[
            jax.ShapeDtypeStruct((nblk, 1, _BLK), jnp.int32),
            jax.ShapeDtypeStruct((b, d), jnp.float32),
        ],
    )(embedding, codebook)
    return idx.reshape(b), alpha


def _make_sc_gather(b, d):
    nw = _NC * _NS
    rpw = b // nw
    mesh = plsc.VectorSubcoreMesh(core_axis_name="c", subcore_axis_name="s")

    @functools.partial(
        pl.kernel, mesh=mesh,
        compiler_params=pltpu.CompilerParams(use_tc_tiling_on_sc=False),
        out_type=jax.ShapeDtypeStruct((b, d), jnp.float32),
        scratch_types=[
            pltpu.VMEM((rpw,), jnp.int32),
            pltpu.VMEM((rpw, d), jnp.float32),
            pltpu.VMEM((rpw, d), jnp.float32),
            pltpu.SemaphoreType.DMA,
        ],
    )
    def sc_gather(cb_hbm, idx_hbm, alpha_hbm, out_hbm, idx_v, alpha_v,
                  rows_v, sem):
        wid = lax.axis_index("s") * _NC + lax.axis_index("c")
        base = wid * rpw
        pltpu.sync_copy(idx_hbm.at[pl.ds(base, rpw)], idx_v)
        pltpu.sync_copy(alpha_hbm.at[pl.ds(base, rpw)], alpha_v)
        pltpu.async_copy(cb_hbm.at[idx_v], rows_v, sem).wait()

        def scale_row(r, _):
            for h in range(d // _LANES):
                sl = pl.ds(h * _LANES, _LANES)
                rows_v[r, sl] = rows_v[r, sl] * alpha_v[r, sl]
            return 0

        lax.fori_loop(0, rpw, scale_row, 0)
        pltpu.sync_copy(rows_v, out_hbm.at[pl.ds(base, rpw)])

    return sc_gather


@jax.jit
def kernel(embedding, codebook):
    if embedding.ndim == 1:
        embedding = embedding[None, :]
    b, d = embedding.shape
    idx, alpha = _tc_search(embedding, codebook)
    z = _make_sc_gather(b, d)(codebook, idx, alpha)
    return (z, idx)


# speedup vs baseline: 1.0003x; 1.0003x over previous
import functools

import jax
import jax.numpy as jnp
from jax import lax
from jax.experimental import pallas as pl
from jax.experimental.pallas import tpu as pltpu
from jax.experimental.pallas import tpu_sc as plsc

_BLK = 512
_NC = 2
_NS = 16
_LANES = 16


def _vq_block(emb_ref, cb_ref, idx_ref, alpha_ref):
    d = emb_ref.shape[1]
    e = emb_ref[...]
    c = cb_ref[...]
    k = c.shape[0]

    norms = jnp.sum(c * c, axis=1)
    dots = jax.lax.dot_general(
        e, c, (((1,), (1,)), ((), ())),
        preferred_element_type=jnp.float32,
        precision=jax.lax.Precision.HIGHEST)
    alpha = dots / norms[None, :]
    score = dots * alpha

    m = jnp.max(score, axis=1, keepdims=True)
    kiota = jax.lax.broadcasted_iota(jnp.int32, score.shape, 1)
    idx = jnp.min(jnp.where(score == m, kiota, k), axis=1)

    sel = kiota == idx[:, None]
    alpha_sel = jnp.max(jnp.where(sel, alpha, -jnp.inf), axis=1)

    idx_ref[0, 0, :] = idx
    alpha_ref[...] = jnp.broadcast_to(alpha_sel[:, None], (alpha_sel.shape[0], d))


def _tc_search(embedding, codebook):
    b, d = embedding.shape
    k = codebook.shape[0]
    nblk = b // _BLK
    idx, alpha = pl.pallas_call(
        _vq_block,
        grid=(nblk,),
        in_specs=[
            pl.BlockSpec((_BLK, d), lambda i: (i, 0)),
            pl.BlockSpec((k, d), lambda i: (0, 0)),
        ],
        out_specs=[
            pl.BlockSpec((1, 1, _BLK), lambda i: (i, 0, 0)),
            pl.BlockSpec((_BLK, d), lambda i: (i, 0)),
        ],
        out_shape=[
            jax.ShapeDtypeStruct((nblk, 1, _BLK), jnp.int32),
            jax.ShapeDtypeStruct((b, d), jnp.float32),
        ],
    )(embedding, codebook)
    return idx.reshape(b), alpha


def _make_sc_gather(b, d):
    nw = _NC * _NS
    rpw = b // nw
    mesh = plsc.VectorSubcoreMesh(core_axis_name="c", subcore_axis_name="s")

    @functools.partial(
        pl.kernel, mesh=mesh,
        compiler_params=pltpu.CompilerParams(use_tc_tiling_on_sc=False,
                                             skip_device_barrier=True),
        out_type=jax.ShapeDtypeStruct((b, d), jnp.float32),
        scratch_types=[
            pltpu.VMEM((rpw,), jnp.int32),
            pltpu.VMEM((rpw, d), jnp.float32),
            pltpu.VMEM((rpw, d), jnp.float32),
            pltpu.SemaphoreType.DMA,
        ],
    )
    def sc_gather(cb_hbm, idx_hbm, alpha_hbm, out_hbm, idx_v, alpha_v,
                  rows_v, sem):
        wid = lax.axis_index("s") * _NC + lax.axis_index("c")
        base = wid * rpw
        pltpu.sync_copy(idx_hbm.at[pl.ds(base, rpw)], idx_v)
        pltpu.sync_copy(alpha_hbm.at[pl.ds(base, rpw)], alpha_v)
        pltpu.async_copy(cb_hbm.at[idx_v], rows_v, sem).wait()

        def scale_row(r, _):
            for h in range(d // _LANES):
                sl = pl.ds(h * _LANES, _LANES)
                rows_v[r, sl] = rows_v[r, sl] * alpha_v[r, sl]
            return 0

        lax.fori_loop(0, rpw, scale_row, 0)
        pltpu.sync_copy(rows_v, out_hbm.at[pl.ds(base, rpw)])

    return sc_gather


@jax.jit
def kernel(embedding, codebook):
    if embedding.ndim == 1:
        embedding = embedding[None, :]
    b, d = embedding.shape
    idx, alpha = _tc_search(embedding, codebook)
    z = _make_sc_gather(b, d)(codebook, idx, alpha)
    return (z, idx)


# speedup vs baseline: 2.1449x; 2.1443x over previous
import functools

import jax
import jax.numpy as jnp
from jax.experimental import pallas as pl

_BLK = 1024


def _vq_block(emb_ref, cb_ref, z_ref, idx_ref):
    e = emb_ref[...]
    c = cb_ref[...]
    k = c.shape[0]

    norms = jnp.sum(c * c, axis=1)
    dots = jax.lax.dot_general(
        e, c, (((1,), (1,)), ((), ())),
        preferred_element_type=jnp.float32,
        precision=jax.lax.Precision.HIGHEST)
    alpha = dots / norms[None, :]
    score = dots * alpha

    m = jnp.max(score, axis=1, keepdims=True)
    kiota = jax.lax.broadcasted_iota(jnp.int32, score.shape, 1)
    idx = jnp.min(jnp.where(score == m, kiota, k), axis=1)

    onehot = (kiota == idx[:, None]).astype(jnp.float32)
    z = jax.lax.dot_general(
        onehot * alpha, c, (((1,), (0,)), ((), ())),
        preferred_element_type=jnp.float32)

    z_ref[...] = z
    idx_ref[0, 0, :] = idx


@functools.partial(jax.jit, static_argnames=())
def kernel(embedding, codebook):
    if embedding.ndim == 1:
        embedding = embedding[None, :]
    b, d = embedding.shape
    k = codebook.shape[0]
    nblk = b // _BLK

    z, idx = pl.pallas_call(
        _vq_block,
        grid=(nblk,),
        in_specs=[
            pl.BlockSpec((_BLK, d), lambda i: (i, 0)),
            pl.BlockSpec((k, d), lambda i: (0, 0)),
        ],
        out_specs=[
            pl.BlockSpec((_BLK, d), lambda i: (i, 0)),
            pl.BlockSpec((1, 1, _BLK), lambda i: (i, 0, 0)),
        ],
        out_shape=[
            jax.ShapeDtypeStruct((b, d), jnp.float32),
            jax.ShapeDtypeStruct((nblk, 1, _BLK), jnp.int32),
        ],
    )(embedding, codebook)
    return (z, idx.reshape(b))


# speedup vs baseline: 2.1704x; 1.0119x over previous
import functools

import jax
import jax.numpy as jnp
from jax.experimental import pallas as pl

_BLK = 2048


def _vq_block(emb_ref, cb_ref, z_ref, idx_ref):
    e = emb_ref[...]
    c = cb_ref[...]
    k = c.shape[0]

    norms = jnp.sum(c * c, axis=1)
    dots = jax.lax.dot_general(
        e, c, (((1,), (1,)), ((), ())),
        preferred_element_type=jnp.float32,
        precision=jax.lax.Precision.HIGHEST)
    alpha = dots / norms[None, :]
    score = dots * alpha

    m = jnp.max(score, axis=1, keepdims=True)
    kiota = jax.lax.broadcasted_iota(jnp.int32, score.shape, 1)
    idx = jnp.min(jnp.where(score == m, kiota, k), axis=1)

    onehot = (kiota == idx[:, None]).astype(jnp.float32)
    z = jax.lax.dot_general(
        onehot * alpha, c, (((1,), (0,)), ((), ())),
        preferred_element_type=jnp.float32)

    z_ref[...] = z
    idx_ref[0, 0, :] = idx


@functools.partial(jax.jit, static_argnames=())
def kernel(embedding, codebook):
    if embedding.ndim == 1:
        embedding = embedding[None, :]
    b, d = embedding.shape
    k = codebook.shape[0]
    nblk = b // _BLK

    z, idx = pl.pallas_call(
        _vq_block,
        grid=(nblk,),
        in_specs=[
            pl.BlockSpec((_BLK, d), lambda i: (i, 0)),
            pl.BlockSpec((k, d), lambda i: (0, 0)),
        ],
        out_specs=[
            pl.BlockSpec((_BLK, d), lambda i: (i, 0)),
            pl.BlockSpec((1, 1, _BLK), lambda i: (i, 0, 0)),
        ],
        out_shape=[
            jax.ShapeDtypeStruct((b, d), jnp.float32),
            jax.ShapeDtypeStruct((nblk, 1, _BLK), jnp.int32),
        ],
    )(embedding, codebook)
    return (z, idx.reshape(b))
